# baseline (speedup 1.0000x reference)
import math

import jax
import jax.numpy as jnp
from jax.experimental import pallas as pl
from jax.experimental.pallas import tpu as pltpu

D_MODEL = 256
GRID_DIM = 30
HW = GRID_DIM * GRID_DIM
G = 9
GHW = G * HW
SPLIT = 3
CHUNK = GHW // SPLIT


def _body(x_ref, io_ref, pair_ref, o_ref, pos_scr, ge_scr):
    b = pl.program_id(0)

    @pl.when(b == 0)
    def _init():
        dim = D_MODEL // 2
        r = jax.lax.broadcasted_iota(jnp.int32, (HW, 2 * dim), 0)
        c = jax.lax.broadcasted_iota(jnp.int32, (HW, 2 * dim), 1)
        pos = jnp.where(c < dim, r // GRID_DIM, r % GRID_DIM).astype(jnp.float32)
        cl = c % dim
        freq = jnp.exp((cl - cl % 2).astype(jnp.float32) * (-math.log(10000.0) / dim))
        angle = pos * freq
        pos_scr[...] = jnp.where(cl % 2 == 0, jnp.sin(angle), jnp.cos(angle))
        for gg in range(G):
            ge_scr[gg, 0:64] = io_ref[gg % 2, :]
            ge_scr[gg, 64:128] = pair_ref[gg // 2, :]

    o_ref[:, :, 0:256] = x_ref[:, :, 0:256] + pos_scr[...][None, :, :]
    o_ref[:, :, 256:384] = x_ref[:, :, 256:384] + ge_scr[...][:, None, :]


@jax.jit
def kernel(x, io_table, pair_table):
    B, Gd, H, W, C = x.shape
    xf = x.reshape(B * Gd, H * W, C)
    out = pl.pallas_call(
        _body,
        grid=(B,),
        in_specs=[
            pl.BlockSpec((Gd, H * W, C), lambda b: (b, 0, 0)),
            pl.BlockSpec(memory_space=pltpu.VMEM),
            pl.BlockSpec(memory_space=pltpu.VMEM),
        ],
        out_specs=pl.BlockSpec((Gd, H * W, C), lambda b: (b, 0, 0)),
        out_shape=jax.ShapeDtypeStruct((B * Gd, H * W, C), x.dtype),
        scratch_shapes=[
            pltpu.VMEM((HW, D_MODEL), jnp.float32),
            pltpu.VMEM((G, D_MODEL // 2), jnp.float32),
        ],
    )(xf, io_table, pair_table)
    return out.reshape(B, Gd, H, W, C)
